# Initial kernel scaffold; baseline (speedup 1.0000x reference)
import jax
import jax.numpy as jnp
from jax.experimental import pallas as pl


def kernel(pos, mass_center, scaler, vector, batch_index, W1n, b1n, W2n, b2n, W1g, b1g, W2g, b2g, Wf, bf):
    raise NotImplementedError("write your pallas kernel here")


# speedup vs baseline: 3.4984x; 3.4984x over previous
import functools

import jax
import jax.numpy as jnp
from jax import lax
from jax.experimental import pallas as pl
from jax.experimental.pallas import tpu as pltpu
from jax.experimental.pallas import tpu_sc as plsc

N = 100000
D = 128
H = 64
G = 512

NW = 32
ROWS_MAIN = 3136
ROWS_LAST = N - (NW - 1) * ROWS_MAIN
CHUNK = 32
ACC_W = 144
N_PAD = NW * ROWS_MAIN

TILE = 2000
NT = N // TILE


def _sc_segment_partials(scaler, bi_pad):
    mesh = plsc.VectorSubcoreMesh(core_axis_name="c", subcore_axis_name="s")

    @functools.partial(
        pl.kernel,
        mesh=mesh,
        compiler_params=pltpu.CompilerParams(use_tc_tiling_on_sc=False,
                                             needs_layout_passes=False),
        out_type=jax.ShapeDtypeStruct((NW, G, ACC_W), jnp.float32),
        scratch_types=[
            pltpu.VMEM((G, ACC_W), jnp.float32),
            pltpu.VMEM((ROWS_MAIN,), jnp.int32),
            pltpu.VMEM((CHUNK, D), jnp.float32),
            pltpu.SemaphoreType.DMA,
        ],
    )
    def k(scaler_hbm, bi_hbm, out_hbm, acc, idxv, buf, sem):
        wid = lax.axis_index("c") * 16 + lax.axis_index("s")
        base = wid * ROWS_MAIN
        trips = jnp.where(wid == NW - 1, ROWS_LAST // CHUNK, ROWS_MAIN // CHUNK)

        pltpu.sync_copy(bi_hbm.at[pl.ds(base, ROWS_MAIN)], idxv)

        iota16 = lax.iota(jnp.int32, 16)
        zeros16 = jnp.zeros((16,), jnp.float32)
        cnt_vec = jnp.where(iota16 == 0, 1.0, 0.0).astype(jnp.float32)
        col_ids = [c * 16 + iota16 for c in range(ACC_W // 16)]

        def zero_body(r, carry):
            for c in range(ACC_W // 16):
                acc[r, pl.ds(c * 16, 16)] = zeros16
            return carry

        lax.fori_loop(0, G, zero_body, 0)

        def chunk_body(gc, carry):
            row0 = base + gc * CHUNK
            pltpu.async_copy(scaler_hbm.at[pl.ds(row0, CHUNK)], buf, sem).wait()
            for r16 in range(CHUNK // 16):
                gvec = idxv[pl.ds(gc * CHUNK + r16 * 16, 16)]
                for r in range(16):
                    row = r16 * 16 + r
                    gs = lax.gather(
                        gvec, jnp.full((16, 1), r, jnp.int32),
                        lax.GatherDimensionNumbers(
                            offset_dims=(), collapsed_slice_dims=(0,),
                            start_index_map=(0,)),
                        (1,),
                        mode=lax.GatherScatterMode.PROMISE_IN_BOUNDS)
                    for c in range(D // 16):
                        vals = buf[row, pl.ds(c * 16, 16)]
                        plsc.addupdate_scatter(acc, [gs, col_ids[c]], vals)
                    plsc.addupdate_scatter(acc, [gs, col_ids[D // 16]], cnt_vec)
            return carry

        lax.fori_loop(0, trips, chunk_body, 0)
        pltpu.sync_copy(acc, out_hbm.at[wid])

    return k(scaler, bi_pad)


def _tc_node_zseg(scaler, bi3d, W1n, b1n, W2n, b2n, wfn):
    def body(x_ref, bi_ref, w1_ref, b1_ref, w2_ref, b2_ref, wf_ref,
             out_ref, zacc):
        pid = pl.program_id(0)

        @pl.when(pid == 0)
        def _():
            zacc[...] = jnp.zeros_like(zacc)

        x = x_ref[...]
        h = x @ w1_ref[...] + b1_ref[...]
        h = h * jax.nn.sigmoid(h)
        y = h @ w2_ref[...] + b2_ref[...]
        y = y * jax.nn.sigmoid(y)
        z = y @ wf_ref[...]
        idx = bi_ref[0]
        gcol = lax.broadcasted_iota(jnp.int32, (G, TILE), 0)
        oht = (gcol == idx).astype(jnp.float32)
        zacc[...] += oht @ z

        @pl.when(pid == NT - 1)
        def _():
            out_ref[...] = zacc[...]

    return pl.pallas_call(
        body,
        grid=(NT,),
        in_specs=[
            pl.BlockSpec((TILE, D), lambda i: (i, 0)),
            pl.BlockSpec((1, 1, TILE), lambda i: (i, 0, 0)),
            pl.BlockSpec((D, H), lambda i: (0, 0)),
            pl.BlockSpec((1, H), lambda i: (0, 0)),
            pl.BlockSpec((H, H), lambda i: (0, 0)),
            pl.BlockSpec((1, H), lambda i: (0, 0)),
            pl.BlockSpec((H, 1), lambda i: (0, 0)),
        ],
        out_specs=pl.BlockSpec((G, 1), lambda i: (0, 0)),
        out_shape=jax.ShapeDtypeStruct((G, 1), jnp.float32),
        scratch_shapes=[pltpu.VMEM((G, 1), jnp.float32)],
    )(scaler, bi3d, W1n, b1n, W2n, b2n, wfn)


def _tc_epilogue(partials, zseg, W1g, b1g, W2g, b2g, wfg, bf):
    def body(p_ref, z_ref, w1_ref, b1_ref, w2_ref, b2_ref, wf_ref, bf_ref,
             out_ref):
        s = jnp.sum(p_ref[...], axis=0)
        sums = s[:, :D]
        counts = s[:, D:D + 1]
        mean = sums / jnp.maximum(counts, 1.0)
        g1 = mean @ w1_ref[...] + b1_ref[...]
        g1 = g1 * jax.nn.sigmoid(g1)
        g2 = g1 @ w2_ref[...] + b2_ref[...]
        g2 = g2 * jax.nn.sigmoid(g2)
        out_ref[...] = g2 @ wf_ref[...] + z_ref[...] + bf_ref[...]

    return pl.pallas_call(
        body,
        out_shape=jax.ShapeDtypeStruct((G, 1), jnp.float32),
    )(partials, zseg, W1g, b1g, W2g, b2g, wfg, bf)


def kernel(pos, mass_center, scaler, vector, batch_index,
           W1n, b1n, W2n, b2n, W1g, b1g, W2g, b2g, Wf, bf):
    bi32 = batch_index.astype(jnp.int32)
    bi_pad = jnp.concatenate([bi32, jnp.zeros((N_PAD - N,), jnp.int32)])
    partials = _sc_segment_partials(scaler, bi_pad)
    zseg = _tc_node_zseg(scaler, bi32.reshape(NT, 1, TILE),
                         W1n, b1n.reshape(1, H), W2n, b2n.reshape(1, H),
                         Wf[H:, :])
    out = _tc_epilogue(partials, zseg, W1g, b1g.reshape(1, H),
                       W2g, b2g.reshape(1, H), Wf[:H, :], bf.reshape(1, 1))
    return out
